# baseline (speedup 1.0000x reference)
import functools

import jax
import jax.numpy as jnp
from jax import lax
from jax.experimental import pallas as pl
from jax.experimental.pallas import tpu as pltpu
from jax.experimental.pallas import tpu_sc as plsc

B, L, E, V = 4096, 200, 16, 1000000
H1, H2, OUT = 200, 80, 2

NW = 32


POS = B * L
POS_PER_W = POS // NW
PCHUNK = 800
PNCH = POS_PER_W // PCHUNK
IB_PER_W = B // NW


def _gather_tec(ig_hbm, is_hbm, ic_hbm, vg_hbm, vs_hbm, vc_hbm, table_hbm,
                out_item, out_series, idx_v, rows_v, sem):
    wid = lax.axis_index("s") * 2 + lax.axis_index("c")
    vsrc = (vg_hbm, vs_hbm, vc_hbm)

    it_base = wid * IB_PER_W
    for f, src in enumerate((ig_hbm, is_hbm, ic_hbm)):
        pltpu.sync_copy(src.at[pl.ds(it_base, IB_PER_W)],
                        idx_v.at[0, f, pl.ds(0, IB_PER_W)])
        pltpu.async_copy(table_hbm.at[idx_v.at[0, f, pl.ds(0, IB_PER_W)]],
                         rows_v.at[0, f, pl.ds(0, IB_PER_W)], sem).wait()
        pltpu.sync_copy(rows_v.at[0, f, pl.ds(0, IB_PER_W)],
                        out_item.at[pl.ds(it_base, IB_PER_W), f])

    def fire(c, s):
        off = wid * POS_PER_W + c * PCHUNK
        for f in range(3):
            pltpu.sync_copy(vsrc[f].at[pl.ds(off, PCHUNK)], idx_v.at[s, f])
            pltpu.async_copy(table_hbm.at[idx_v.at[s, f]],
                             rows_v.at[s, f], sem)

    def drain_write(c, s):
        off = wid * POS_PER_W + c * PCHUNK
        for f in range(3):
            pltpu.make_async_copy(table_hbm.at[idx_v.at[s, f]],
                                  rows_v.at[s, f], sem).wait()
            pltpu.sync_copy(rows_v.at[s, f],
                            out_series.at[pl.ds(off, PCHUNK), f])

    fire(0, 0)

    def body(i, carry):
        c0 = 2 * i
        fire(c0 + 1, 1)
        drain_write(c0, 0)

        @pl.when(c0 + 2 < PNCH)
        def _fire_next():
            fire(c0 + 2, 0)

        drain_write(c0 + 1, 1)
        return carry

    lax.fori_loop(0, PNCH // 2, body, 0)


def _sc_gather(ig, i_s, ic, vg, vs, vc, table):
    mesh = plsc.VectorSubcoreMesh(core_axis_name="c", subcore_axis_name="s")
    f = functools.partial(
        pl.kernel, mesh=mesh,
        compiler_params=pltpu.CompilerParams(use_tc_tiling_on_sc=False),
        out_type=(
            jax.ShapeDtypeStruct((B, 3, E), jnp.float32),
            jax.ShapeDtypeStruct((POS, 3, E), jnp.float32),
        ),
        scratch_types=[
            pltpu.VMEM((2, 3, PCHUNK), jnp.int32),
            pltpu.VMEM((2, 3, PCHUNK, E), jnp.float32),
            pltpu.SemaphoreType.DMA,
        ],
    )(_gather_tec)
    return f(ig, i_s, ic, vg, vs, vc, table)


def _attn_body(vgt_ref, xit_ref, xs_ref, pooled_ref, xst_ref):
    j = pl.program_id(1)
    xs2 = xs_ref[...]
    xst2 = jnp.transpose(xs2, (1, 0))
    xst = xst2.reshape(-1, 3 * E, xst2.shape[-1])
    xst_ref[...] = xst
    xi_t = xit_ref[...]
    scores_t = jnp.sum(xst * xi_t[None, :, :], axis=1)
    maskf_t = (vgt_ref[...] != 0).astype(jnp.float32)
    ms_t = scores_t * maskf_t
    part_t = jnp.sum(xst * ms_t[:, None, :], axis=0)

    @pl.when(j == 0)
    def _init():
        pooled_ref[...] = part_t

    @pl.when(j != 0)
    def _acc():
        pooled_ref[...] += part_t


def _attention(vg_t, x_item_t, x_series_flat, bb=512, lb=40):
    grid = (B // bb, L // lb)
    return pl.pallas_call(
        _attn_body,
        grid=grid,
        in_specs=[
            pl.BlockSpec((lb, bb), lambda i, j: (j, i)),
            pl.BlockSpec((3 * E, bb), lambda i, j: (0, i)),
            pl.BlockSpec((bb, lb * 3 * E), lambda i, j: (i, j)),
        ],
        out_specs=[
            pl.BlockSpec((3 * E, bb), lambda i, j: (0, i)),
            pl.BlockSpec((lb, 3 * E, bb), lambda i, j: (j, 0, i)),
        ],
        out_shape=[
            jax.ShapeDtypeStruct((3 * E, B), jnp.float32),
            jax.ShapeDtypeStruct((L, 3 * E, B), jnp.float32),
        ],
    )(vg_t, x_item_t, x_series_flat)


def _sigmoid(x):
    return 1.0 / (1.0 + jnp.exp(-x))


def _ln(x, gamma, beta, eps=1e-3):
    mu = jnp.mean(x, axis=-1, keepdims=True)
    var = jnp.mean((x - mu) ** 2, axis=-1, keepdims=True)
    return gamma * (x - mu) / jnp.sqrt(var + eps) + beta


def _dice_act(x, alpha, eps=1e-3):
    mu = jnp.mean(x, axis=0, keepdims=True)
    var = jnp.mean((x - mu) ** 2, axis=0, keepdims=True)
    xn = (x - mu) / jnp.sqrt(var + eps)
    p = _sigmoid(xn)
    return alpha * (1.0 - p) * x + p * x


def _mlp_body(xi_ref, pooled_ref, w1a_ref, w1b_ref, b1_ref, g1_ref, be1_ref,
              a1_ref, w2_ref, b2_ref, g2_ref, be2_ref, a2_ref, w3_ref, b3_ref,
              out_ref):
    h = (jnp.dot(xi_ref[...], w1a_ref[...],
                 preferred_element_type=jnp.float32)
         + lax.dot_general(pooled_ref[...], w1b_ref[...],
                           (((0,), (0,)), ((), ())),
                           preferred_element_type=jnp.float32)
         + b1_ref[...])
    h = _ln(h, g1_ref[...], be1_ref[...])
    h = _dice_act(h, a1_ref[...])
    h = jnp.dot(h, w2_ref[...], preferred_element_type=jnp.float32) + b2_ref[...]
    h = _ln(h, g2_ref[...], be2_ref[...])
    h = _dice_act(h, a2_ref[...])
    logits = jnp.dot(h, w3_ref[...], preferred_element_type=jnp.float32) + b3_ref[...]
    m = jnp.max(logits, axis=-1, keepdims=True)
    e = jnp.exp(logits - m)
    out_ref[...] = e / jnp.sum(e, axis=-1, keepdims=True)


def _mlp(x_item, pooled_t, W1, b1, g1, be1, a1, W2, b2, g2, be2, a2, W3, b3):
    args = (x_item, pooled_t, W1[:3 * E], W1[3 * E:], b1.reshape(1, -1),
            g1.reshape(1, -1),
            be1.reshape(1, -1), a1.reshape(1, -1), W2, b2.reshape(1, -1),
            g2.reshape(1, -1), be2.reshape(1, -1), a2.reshape(1, -1),
            W3, b3.reshape(1, -1))
    return pl.pallas_call(
        _mlp_body,
        out_shape=jax.ShapeDtypeStruct((B, OUT), jnp.float32),
    )(*args)


def kernel(i_goods_id, i_shop_id, i_cate_id, visited_goods_ids,
           visited_shop_ids, visited_cate_ids, emb_table, W1, b1, g1, be1, a1,
           W2, b2, g2, be2, a2, W3, b3):
    item_rows, series_rows = _sc_gather(
        i_goods_id, i_shop_id, i_cate_id,
        visited_goods_ids.reshape(-1), visited_shop_ids.reshape(-1),
        visited_cate_ids.reshape(-1), emb_table)
    x_item = item_rows.reshape(B, 3 * E)
    x_series_flat = series_rows.reshape(B, L * 3 * E)
    valid_mask = visited_goods_ids != 0

    (pooled_t, xs_t) = _attention(
        jnp.transpose(visited_goods_ids), jnp.transpose(x_item),
        x_series_flat)
    x_series_out = jnp.transpose(xs_t, (2, 0, 1))
    output = _mlp(x_item, pooled_t, W1, b1, g1, be1, a1,
                  W2, b2, g2, be2, a2, W3, b3)
    return output, x_series_out, valid_mask
